# baseline (speedup 1.0000x reference)
import jax
import jax.numpy as jnp
from jax import lax
from jax.experimental import pallas as pl
from jax.experimental.pallas import tpu as pltpu
from jax.experimental.pallas import tpu_sc as plsc

_N = 1048576
_GRID = 16
_NC = 2
_NS = 16
_L = 16
_NW = _NC * _NS
_CPW = _N // _NW
_CH = 8192
_NCH = _CPW // _CH
_TBL = _GRID * _GRID * 3
_PLANE = _GRID * _GRID


def _tile_body(a_hbm, b_hbm, coeffs_hbm, scale_hbm, out_hbm,
               tq_v, big_v, sc_v,
               a0_v, a1_v, b0_v, b1_v, o0_v, o1_v,
               sa0, sa1, sb0, sb1, so0, so1):
    wid = lax.axis_index("s") * _NC + lax.axis_index("c")
    base = wid * _CPW
    aa = [a0_v, a1_v]
    bb = [b0_v, b1_v]
    oo = [o0_v, o1_v]
    sa = [sa0, sa1]
    sb = [sb0, sb1]
    so = [so0, so1]

    ha = [None] * _NCH
    hb = [None] * _NCH
    ha[0] = pltpu.async_copy(a_hbm.at[pl.ds(base, _CH)], aa[0], sa[0])
    hb[0] = pltpu.async_copy(b_hbm.at[pl.ds(base, _CH)], bb[0], sb[0])

    pltpu.sync_copy(coeffs_hbm, tq_v.at[pl.ds(0, _TBL)])
    pltpu.sync_copy(scale_hbm, sc_v)
    sv = sc_v[...]

    def prep(i, carry):
        c = tq_v[pl.ds(i * _L, _L)]
        q = jnp.where(c > 0.3, 1.0, jnp.where(c < -0.3, -1.0, 0.0))
        tq_v[pl.ds(i * _L, _L)] = (c + (q - c)) * sv
        return carry

    lax.fori_loop(0, _TBL // _L, prep, 0)

    @plsc.parallel_loop(0, _TBL, unroll=8)
    def repl(r):
        p = r >> 8
        fl = r & (_PLANE - 1)
        val = tq_v[pl.ds(fl * 3 + p, _L)][0]
        big_v[pl.ds(r * _L, _L)] = jnp.broadcast_to(val, (_L,))

    lane = lax.iota(jnp.int32, _L)

    def compute_chunk(a_v, b_v, o_v):
        @plsc.parallel_loop(0, _CH // _L, unroll=6)
        def body(vi):
            off = vi * _L
            av = a_v[pl.ds(off, _L)]
            bv = b_v[pl.ds(off, _L)]
            xa = (av + 1.0) * 8.0
            xb = (bv + 1.0) * 8.0
            ia = jnp.minimum(jnp.maximum(xa.astype(jnp.int32), 0), _GRID - 1)
            ib = jnp.minimum(jnp.maximum(xb.astype(jnp.int32), 0), _GRID - 1)
            flat = (ia << 8) + (ib << 4) + lane
            q0 = plsc.load_gather(big_v, [flat])
            q1 = plsc.load_gather(big_v, [flat + _PLANE * _L])
            q2 = plsc.load_gather(big_v, [flat + 2 * _PLANE * _L])
            la = xa - ia.astype(jnp.float32)
            lb = xb - ib.astype(jnp.float32)
            o_v[pl.ds(off, _L)] = q0 + q1 * la + q2 * lb

    ho = [None, None]
    for k in range(_NCH):
        j = k % 2
        ha[k].wait()
        hb[k].wait()
        if k + 1 < _NCH:
            nj = (k + 1) % 2
            off = base + (k + 1) * _CH
            ha[k + 1] = pltpu.async_copy(a_hbm.at[pl.ds(off, _CH)], aa[nj], sa[nj])
            hb[k + 1] = pltpu.async_copy(b_hbm.at[pl.ds(off, _CH)], bb[nj], sb[nj])
        if ho[j] is not None:
            ho[j].wait()
        compute_chunk(aa[j], bb[j], oo[j])
        ho[j] = pltpu.async_copy(oo[j], out_hbm.at[pl.ds(base + k * _CH, _CH)], so[j])
    for h in ho:
        if h is not None:
            h.wait()


def kernel(a, b, coeffs, scale):
    mesh = plsc.VectorSubcoreMesh(core_axis_name="c", subcore_axis_name="s")
    run = pl.kernel(
        _tile_body,
        mesh=mesh,
        compiler_params=pltpu.CompilerParams(needs_layout_passes=False),
        out_type=jax.ShapeDtypeStruct((_N,), jnp.float32),
        scratch_types=[
            pltpu.VMEM((_TBL + _L,), jnp.float32),
            pltpu.VMEM((_TBL * _L,), jnp.float32),
            pltpu.VMEM((_L,), jnp.float32),
            pltpu.VMEM((_CH,), jnp.float32),
            pltpu.VMEM((_CH,), jnp.float32),
            pltpu.VMEM((_CH,), jnp.float32),
            pltpu.VMEM((_CH,), jnp.float32),
            pltpu.VMEM((_CH,), jnp.float32),
            pltpu.VMEM((_CH,), jnp.float32),
            pltpu.SemaphoreType.DMA,
            pltpu.SemaphoreType.DMA,
            pltpu.SemaphoreType.DMA,
            pltpu.SemaphoreType.DMA,
            pltpu.SemaphoreType.DMA,
            pltpu.SemaphoreType.DMA,
        ],
    )
    coeffs_flat = coeffs.reshape(_TBL)
    scale_vec = jnp.broadcast_to(scale, (_L,))
    return run(a, b, coeffs_flat, scale_vec)


# speedup vs baseline: 1.0339x; 1.0339x over previous
import jax
import jax.numpy as jnp
from jax import lax
from jax.experimental import pallas as pl
from jax.experimental.pallas import tpu as pltpu
from jax.experimental.pallas import tpu_sc as plsc

_N = 1048576
_GRID = 16
_NC = 2
_NS = 16
_L = 16
_NW = _NC * _NS
_CPW = _N // _NW
_CH = 16384
_NCH = _CPW // _CH
_TBL = _GRID * _GRID * 3
_PLANE = _GRID * _GRID


def _tile_body(a_hbm, b_hbm, coeffs_hbm, scale_hbm, out_hbm,
               tq_v, big_v, sc_v,
               a0_v, a1_v, b0_v, b1_v, o0_v, o1_v,
               sa0, sa1, sb0, sb1, so0, so1):
    wid = lax.axis_index("s") * _NC + lax.axis_index("c")
    base = wid * _CPW
    aa = [a0_v, a1_v]
    bb = [b0_v, b1_v]
    oo = [o0_v, o1_v]
    sa = [sa0, sa1]
    sb = [sb0, sb1]
    so = [so0, so1]

    ha = [None] * _NCH
    hb = [None] * _NCH
    ha[0] = pltpu.async_copy(a_hbm.at[pl.ds(base, _CH)], aa[0], sa[0])
    hb[0] = pltpu.async_copy(b_hbm.at[pl.ds(base, _CH)], bb[0], sb[0])

    pltpu.sync_copy(coeffs_hbm, tq_v.at[pl.ds(0, _TBL)])
    pltpu.sync_copy(scale_hbm, sc_v)
    sv = sc_v[...]

    def prep(i, carry):
        c = tq_v[pl.ds(i * _L, _L)]
        q = jnp.where(c > 0.3, 1.0, jnp.where(c < -0.3, -1.0, 0.0))
        tq_v[pl.ds(i * _L, _L)] = (c + (q - c)) * sv
        return carry

    lax.fori_loop(0, _TBL // _L, prep, 0)

    @plsc.parallel_loop(0, _TBL, unroll=8)
    def repl(r):
        p = r >> 8
        fl = r & (_PLANE - 1)
        val = tq_v[pl.ds(fl * 3 + p, _L)][0]
        big_v[pl.ds(r * _L, _L)] = jnp.broadcast_to(val, (_L,))

    lane = lax.iota(jnp.int32, _L)

    def compute_chunk(a_v, b_v, o_v):
        @plsc.parallel_loop(0, _CH // _L, unroll=4)
        def body(vi):
            off = vi * _L
            av = a_v[pl.ds(off, _L)]
            bv = b_v[pl.ds(off, _L)]
            xa = (av + 1.0) * 8.0
            xb = (bv + 1.0) * 8.0
            ia = jnp.minimum(jnp.maximum(xa.astype(jnp.int32), 0), _GRID - 1)
            ib = jnp.minimum(jnp.maximum(xb.astype(jnp.int32), 0), _GRID - 1)
            flat = (ia << 8) + (ib << 4) + lane
            q0 = plsc.load_gather(big_v, [flat])
            q1 = plsc.load_gather(big_v, [flat + _PLANE * _L])
            q2 = plsc.load_gather(big_v, [flat + 2 * _PLANE * _L])
            la = xa - ia.astype(jnp.float32)
            lb = xb - ib.astype(jnp.float32)
            o_v[pl.ds(off, _L)] = q0 + q1 * la + q2 * lb

    ho = [None, None]
    for k in range(_NCH):
        j = k % 2
        ha[k].wait()
        hb[k].wait()
        if k + 1 < _NCH:
            nj = (k + 1) % 2
            off = base + (k + 1) * _CH
            ha[k + 1] = pltpu.async_copy(a_hbm.at[pl.ds(off, _CH)], aa[nj], sa[nj])
            hb[k + 1] = pltpu.async_copy(b_hbm.at[pl.ds(off, _CH)], bb[nj], sb[nj])
        if ho[j] is not None:
            ho[j].wait()
        compute_chunk(aa[j], bb[j], oo[j])
        ho[j] = pltpu.async_copy(oo[j], out_hbm.at[pl.ds(base + k * _CH, _CH)], so[j])
    for h in ho:
        if h is not None:
            h.wait()


def kernel(a, b, coeffs, scale):
    mesh = plsc.VectorSubcoreMesh(core_axis_name="c", subcore_axis_name="s")
    run = pl.kernel(
        _tile_body,
        mesh=mesh,
        compiler_params=pltpu.CompilerParams(needs_layout_passes=False),
        out_type=jax.ShapeDtypeStruct((_N,), jnp.float32),
        scratch_types=[
            pltpu.VMEM((_TBL + _L,), jnp.float32),
            pltpu.VMEM((_TBL * _L,), jnp.float32),
            pltpu.VMEM((_L,), jnp.float32),
            pltpu.VMEM((_CH,), jnp.float32),
            pltpu.VMEM((_CH,), jnp.float32),
            pltpu.VMEM((_CH,), jnp.float32),
            pltpu.VMEM((_CH,), jnp.float32),
            pltpu.VMEM((_CH,), jnp.float32),
            pltpu.VMEM((_CH,), jnp.float32),
            pltpu.SemaphoreType.DMA,
            pltpu.SemaphoreType.DMA,
            pltpu.SemaphoreType.DMA,
            pltpu.SemaphoreType.DMA,
            pltpu.SemaphoreType.DMA,
            pltpu.SemaphoreType.DMA,
        ],
    )
    coeffs_flat = coeffs.reshape(_TBL)
    scale_vec = jnp.broadcast_to(scale, (_L,))
    return run(a, b, coeffs_flat, scale_vec)
